# baseline (speedup 1.0000x reference)
import functools

import jax
import jax.numpy as jnp
from jax import lax
from jax.experimental import pallas as pl
from jax.experimental.pallas import tpu as pltpu
from jax.experimental.pallas import tpu_sc as plsc

B, S, H = 2, 2048, 2048
E = 8
TOPK = 2
I = 1024
N = B * S

M = 512
ML = 1024

NC, NS, L = 2, 16, 16
NW = NC * NS
TPW = N // NW
G = TPW // L


def _logits_body(x_ref, wr_ref, out_ref):
    out_ref[...] = jax.lax.dot_general(
        wr_ref[...], x_ref[...], (((1,), (1,)), ((), ())),
        preferred_element_type=jnp.float32)


def _router_sc_body(logits_hbm, w_hbm, lg_v, w_v):
    wid = lax.axis_index("s") * NC + lax.axis_index("c")
    base = wid * TPW
    for e in range(E):
        pltpu.sync_copy(logits_hbm.at[pl.ds(e * N + base, TPW)],
                        lg_v.at[pl.ds(e * TPW, TPW)])
    for g in range(G):
        ls = [lg_v[pl.ds(e * TPW + g * L, L)] for e in range(E)]
        mx = ls[0]
        for e in range(1, E):
            mx = jnp.maximum(mx, ls[e])
        es = [jnp.exp(l - mx) for l in ls]
        v1 = es[0]
        for e in range(1, E):
            v1 = jnp.maximum(v1, es[e])
        neg = jnp.full((L,), -jnp.inf, jnp.float32)
        cnt = jnp.zeros((L,), jnp.float32)
        vex = neg
        for e in range(E):
            cnt = cnt + jnp.where(es[e] == v1, 1.0, 0.0)
            vex = jnp.maximum(vex, jnp.where(es[e] == v1, neg, es[e]))
        v2 = jnp.where(cnt >= 2.0, v1, vex)
        denom = v1 + v2
        w_v[pl.ds(g * L, L)] = v1 / denom
        w_v[pl.ds(TPW + g * L, L)] = v2 / denom
    pltpu.sync_copy(w_v.at[pl.ds(0, TPW)], w_hbm.at[pl.ds(base, TPW)])
    pltpu.sync_copy(w_v.at[pl.ds(TPW, TPW)], w_hbm.at[pl.ds(N + base, TPW)])


_router_sc = functools.partial(
    pl.kernel,
    out_type=jax.ShapeDtypeStruct((TOPK * N,), jnp.float32),
    scratch_types=[
        pltpu.VMEM((E * TPW,), jnp.float32),
        pltpu.VMEM((TOPK * TPW,), jnp.float32),
    ],
    mesh=plsc.VectorSubcoreMesh(core_axis_name="c", subcore_axis_name="s"),
)(_router_sc_body)


def _mlp_body(x_ref, w_ref, wgu_ref, wd_ref, out_ref):
    x16 = x_ref[...].astype(jnp.bfloat16)
    lane = jax.lax.broadcasted_iota(jnp.int32, (M, TOPK), 1)
    acc = None
    for e in range(TOPK):
        gate = jnp.dot(x16, wgu_ref[e, :, :I], preferred_element_type=jnp.float32)
        up = jnp.dot(x16, wgu_ref[e, :, I:], preferred_element_type=jnp.float32)
        w = jnp.sum(jnp.where(lane == e, w_ref[...], 0.0), axis=1,
                    keepdims=True)
        hid = gate * jax.nn.sigmoid(gate) * (up * w)
        part = jnp.dot(hid.astype(jnp.bfloat16), wd_ref[e],
                       preferred_element_type=jnp.float32)
        acc = part if acc is None else acc + part
    out_ref[...] = acc


@jax.jit
def kernel(hidden_states, router_weight, gate_up_weights, down_weights):
    b, s, h = hidden_states.shape
    n = b * s
    hflat = hidden_states.reshape(n, h)

    logits_t = pl.pallas_call(
        _logits_body,
        grid=(n // ML,),
        in_specs=[
            pl.BlockSpec((ML, h), lambda t: (t, 0)),
            pl.BlockSpec((E, h), lambda t: (0, 0)),
        ],
        out_specs=pl.BlockSpec((E, ML), lambda t: (0, t)),
        out_shape=jax.ShapeDtypeStruct((E, n), jnp.float32),
    )(hflat, router_weight)

    w = _router_sc(logits_t.reshape(E * n)).reshape(TOPK, n).T

    gu16 = gate_up_weights[:TOPK].astype(jnp.bfloat16)
    dn16 = down_weights[:TOPK].astype(jnp.bfloat16)

    out = pl.pallas_call(
        _mlp_body,
        grid=(n // M,),
        in_specs=[
            pl.BlockSpec((M, h), lambda t: (t, 0)),
            pl.BlockSpec((M, TOPK), lambda t: (t, 0)),
            pl.BlockSpec((TOPK, h, 2 * I), lambda t: (0, 0, 0)),
            pl.BlockSpec((TOPK, I, h), lambda t: (0, 0, 0)),
        ],
        out_specs=pl.BlockSpec((M, h), lambda t: (t, 0)),
        out_shape=jax.ShapeDtypeStruct((n, h), jnp.float32),
    )(hflat, w, gu16, dn16)

    return out.reshape(b, s, h)


# speedup vs baseline: 1.0020x; 1.0020x over previous
import functools

import jax
import jax.numpy as jnp
from jax import lax
from jax.experimental import pallas as pl
from jax.experimental.pallas import tpu as pltpu
from jax.experimental.pallas import tpu_sc as plsc

B, S, H = 2, 2048, 2048
E = 8
TOPK = 2
I = 1024
N = B * S

M = 512
ML = 1024

NC, NS, L = 2, 16, 16
NW = NC * NS
TPW = N // NW
G = TPW // L


def _logits_body(x_ref, wr_ref, out_ref):
    out_ref[...] = jax.lax.dot_general(
        wr_ref[...], x_ref[...], (((1,), (1,)), ((), ())),
        preferred_element_type=jnp.float32)


def _router_sc_body(logits_hbm, w_hbm, lg_v, w_v):
    wid = lax.axis_index("s") * NC + lax.axis_index("c")
    base = wid * TPW
    for e in range(E):
        pltpu.sync_copy(logits_hbm.at[pl.ds(e * N + base, TPW)],
                        lg_v.at[pl.ds(e * TPW, TPW)])
    for g in range(G):
        ls = [lg_v[pl.ds(e * TPW + g * L, L)] for e in range(E)]
        mx = ls[0]
        for e in range(1, E):
            mx = jnp.maximum(mx, ls[e])
        es = [jnp.exp(l - mx) for l in ls]
        v1 = es[0]
        for e in range(1, E):
            v1 = jnp.maximum(v1, es[e])
        neg = jnp.full((L,), -jnp.inf, jnp.float32)
        cnt = jnp.zeros((L,), jnp.float32)
        vex = neg
        for e in range(E):
            cnt = cnt + jnp.where(es[e] == v1, 1.0, 0.0)
            vex = jnp.maximum(vex, jnp.where(es[e] == v1, neg, es[e]))
        v2 = jnp.where(cnt >= 2.0, v1, vex)
        denom = v1 + v2
        w_v[pl.ds(g * L, L)] = v1 / denom
        w_v[pl.ds(TPW + g * L, L)] = v2 / denom
    pltpu.sync_copy(w_v.at[pl.ds(0, TPW)], w_hbm.at[pl.ds(base, TPW)])
    pltpu.sync_copy(w_v.at[pl.ds(TPW, TPW)], w_hbm.at[pl.ds(N + base, TPW)])


_router_sc = functools.partial(
    pl.kernel,
    out_type=jax.ShapeDtypeStruct((TOPK * N,), jnp.float32),
    scratch_types=[
        pltpu.VMEM((E * TPW,), jnp.float32),
        pltpu.VMEM((TOPK * TPW,), jnp.float32),
    ],
    mesh=plsc.VectorSubcoreMesh(core_axis_name="c", subcore_axis_name="s"),
)(_router_sc_body)


IB = 256


def _mlp_body(x_ref, w_ref, wgu_ref, wd_ref, out_ref):
    x16 = x_ref[...].astype(jnp.bfloat16)
    lane = jax.lax.broadcasted_iota(jnp.int32, (M, TOPK), 1)
    acc = None
    for e in range(TOPK):
        w = jnp.sum(jnp.where(lane == e, w_ref[...], 0.0), axis=1,
                    keepdims=True)
        for c in range(I // IB):
            gate = jnp.dot(x16, wgu_ref[e, :, c * IB:(c + 1) * IB],
                           preferred_element_type=jnp.float32)
            up = jnp.dot(x16, wgu_ref[e, :, I + c * IB:I + (c + 1) * IB],
                         preferred_element_type=jnp.float32)
            hid = gate * jax.nn.sigmoid(gate) * (up * w)
            part = jnp.dot(hid.astype(jnp.bfloat16),
                           wd_ref[e, c * IB:(c + 1) * IB, :],
                           preferred_element_type=jnp.float32)
            acc = part if acc is None else acc + part
    out_ref[...] = acc


@jax.jit
def kernel(hidden_states, router_weight, gate_up_weights, down_weights):
    b, s, h = hidden_states.shape
    n = b * s
    hflat = hidden_states.reshape(n, h)

    logits_t = pl.pallas_call(
        _logits_body,
        grid=(n // ML,),
        in_specs=[
            pl.BlockSpec((ML, h), lambda t: (t, 0)),
            pl.BlockSpec((E, h), lambda t: (0, 0)),
        ],
        out_specs=pl.BlockSpec((E, ML), lambda t: (0, t)),
        out_shape=jax.ShapeDtypeStruct((E, n), jnp.float32),
    )(hflat, router_weight)

    w = _router_sc(logits_t.reshape(E * n)).reshape(TOPK, n).T

    gu16 = gate_up_weights[:TOPK].astype(jnp.bfloat16)
    dn16 = down_weights[:TOPK].astype(jnp.bfloat16)

    out = pl.pallas_call(
        _mlp_body,
        grid=(n // M,),
        in_specs=[
            pl.BlockSpec((M, h), lambda t: (t, 0)),
            pl.BlockSpec((M, TOPK), lambda t: (t, 0)),
            pl.BlockSpec((TOPK, h, 2 * I), lambda t: (0, 0, 0)),
            pl.BlockSpec((TOPK, I, h), lambda t: (0, 0, 0)),
        ],
        out_specs=pl.BlockSpec((M, h), lambda t: (t, 0)),
        out_shape=jax.ShapeDtypeStruct((n, h), jnp.float32),
    )(hflat, w, gu16, dn16)

    return out.reshape(b, s, h)


# speedup vs baseline: 1.0335x; 1.0315x over previous
import functools

import jax
import jax.numpy as jnp
from jax import lax
from jax.experimental import pallas as pl
from jax.experimental.pallas import tpu as pltpu
from jax.experimental.pallas import tpu_sc as plsc

B, S, H = 2, 2048, 2048
E = 8
TOPK = 2
I = 1024
N = B * S

M = 512
ML = 1024

NC, NS, L = 2, 16, 16
NW = NC * NS
TPW = N // NW
G = TPW // L


def _logits_body(x_ref, wr_ref, out_ref):
    out_ref[...] = jax.lax.dot_general(
        wr_ref[...], x_ref[...], (((1,), (1,)), ((), ())),
        preferred_element_type=jnp.float32)


def _router_sc_body(logits_hbm, w_hbm, lg_v, w_v):
    wid = lax.axis_index("s") * NC + lax.axis_index("c")
    base = wid * TPW
    for e in range(E):
        pltpu.sync_copy(logits_hbm.at[pl.ds(e * N + base, TPW)],
                        lg_v.at[pl.ds(e * TPW, TPW)])
    for g in range(G):
        ls = [lg_v[pl.ds(e * TPW + g * L, L)] for e in range(E)]
        mx = ls[0]
        for e in range(1, E):
            mx = jnp.maximum(mx, ls[e])
        es = [jnp.exp(l - mx) for l in ls]
        v1 = es[0]
        for e in range(1, E):
            v1 = jnp.maximum(v1, es[e])
        neg = jnp.full((L,), -jnp.inf, jnp.float32)
        cnt = jnp.zeros((L,), jnp.float32)
        vex = neg
        for e in range(E):
            cnt = cnt + jnp.where(es[e] == v1, 1.0, 0.0)
            vex = jnp.maximum(vex, jnp.where(es[e] == v1, neg, es[e]))
        v2 = jnp.where(cnt >= 2.0, v1, vex)
        denom = v1 + v2
        w_v[pl.ds(g * L, L)] = v1 / denom
        w_v[pl.ds(TPW + g * L, L)] = v2 / denom
    pltpu.sync_copy(w_v.at[pl.ds(0, TPW)], w_hbm.at[pl.ds(base, TPW)])
    pltpu.sync_copy(w_v.at[pl.ds(TPW, TPW)], w_hbm.at[pl.ds(N + base, TPW)])


_router_sc = functools.partial(
    pl.kernel,
    out_type=jax.ShapeDtypeStruct((TOPK * N,), jnp.float32),
    scratch_types=[
        pltpu.VMEM((E * TPW,), jnp.float32),
        pltpu.VMEM((TOPK * TPW,), jnp.float32),
    ],
    mesh=plsc.VectorSubcoreMesh(core_axis_name="c", subcore_axis_name="s"),
)(_router_sc_body)


IB = 256
SCHUNK = 512
DCHUNK = 256


def _mlp_body(x_ref, w_ref, wgu_any, wd_any, out_ref,
              gu16, dn16, sg, sd, sem_g, sem_d):
    GC = 2 * I // SCHUNK
    DC = H // DCHUNK

    @pl.when(pl.program_id(0) == 0)
    def _stage():
        def gu_copy(i, buf):
            e, c = divmod(i, GC)
            return pltpu.make_async_copy(
                wgu_any.at[e, :, pl.ds(c * SCHUNK, SCHUNK)],
                sg.at[buf], sem_g.at[buf])

        def dn_copy(i, buf):
            e, c = divmod(i, DC)
            return pltpu.make_async_copy(
                wd_any.at[e, :, pl.ds(c * DCHUNK, DCHUNK)],
                sd.at[buf], sem_d.at[buf])

        gu_copy(0, 0).start()
        dn_copy(0, 0).start()
        for i in range(TOPK * GC):
            if i + 1 < TOPK * GC:
                gu_copy(i + 1, (i + 1) % 2).start()
            gu_copy(i, i % 2).wait()
            e, c = divmod(i, GC)
            gu16[e, :, c * SCHUNK:(c + 1) * SCHUNK] = (
                sg[i % 2].astype(jnp.bfloat16))
        for i in range(TOPK * DC):
            if i + 1 < TOPK * DC:
                dn_copy(i + 1, (i + 1) % 2).start()
            dn_copy(i, i % 2).wait()
            e, c = divmod(i, DC)
            dn16[e, :, c * DCHUNK:(c + 1) * DCHUNK] = (
                sd[i % 2].astype(jnp.bfloat16))

    x16 = x_ref[...].astype(jnp.bfloat16)
    lane = jax.lax.broadcasted_iota(jnp.int32, (M, TOPK), 1)
    acc = None
    for e in range(TOPK):
        w = jnp.sum(jnp.where(lane == e, w_ref[...], 0.0), axis=1,
                    keepdims=True)
        for c in range(I // IB):
            gate = jnp.dot(x16, gu16[e, :, c * IB:(c + 1) * IB],
                           preferred_element_type=jnp.float32)
            up = jnp.dot(x16, gu16[e, :, I + c * IB:I + (c + 1) * IB],
                         preferred_element_type=jnp.float32)
            hid = gate * jax.nn.sigmoid(gate) * (up * w)
            part = jnp.dot(hid.astype(jnp.bfloat16),
                           dn16[e, c * IB:(c + 1) * IB, :],
                           preferred_element_type=jnp.float32)
            acc = part if acc is None else acc + part
    out_ref[...] = acc


@jax.jit
def kernel(hidden_states, router_weight, gate_up_weights, down_weights):
    b, s, h = hidden_states.shape
    n = b * s
    hflat = hidden_states.reshape(n, h)

    logits_t = pl.pallas_call(
        _logits_body,
        grid=(n // ML,),
        in_specs=[
            pl.BlockSpec((ML, h), lambda t: (t, 0)),
            pl.BlockSpec((E, h), lambda t: (0, 0)),
        ],
        out_specs=pl.BlockSpec((E, ML), lambda t: (0, t)),
        out_shape=jax.ShapeDtypeStruct((E, n), jnp.float32),
    )(hflat, router_weight)

    w = _router_sc(logits_t.reshape(E * n)).reshape(TOPK, n).T

    out = pl.pallas_call(
        _mlp_body,
        grid=(n // M,),
        in_specs=[
            pl.BlockSpec((M, h), lambda t: (t, 0)),
            pl.BlockSpec((M, TOPK), lambda t: (t, 0)),
            pl.BlockSpec(memory_space=pl.ANY),
            pl.BlockSpec(memory_space=pl.ANY),
        ],
        out_specs=pl.BlockSpec((M, h), lambda t: (t, 0)),
        out_shape=jax.ShapeDtypeStruct((n, h), jnp.float32),
        scratch_shapes=[
            pltpu.VMEM((TOPK, h, 2 * I), jnp.bfloat16),
            pltpu.VMEM((TOPK, I, h), jnp.bfloat16),
            pltpu.VMEM((2, h, SCHUNK), jnp.float32),
            pltpu.VMEM((2, I, DCHUNK), jnp.float32),
            pltpu.SemaphoreType.DMA((2,)),
            pltpu.SemaphoreType.DMA((2,)),
        ],
    )(hflat, w, gate_up_weights, down_weights)

    return out.reshape(b, s, h)
